# Initial kernel scaffold; baseline (speedup 1.0000x reference)
import jax
import jax.numpy as jnp
from jax.experimental import pallas as pl


def kernel(beta, actions_onehot, power_states, prev_assigns, W1, b1, W2, b2, W3, b3):
    raise NotImplementedError("write your pallas kernel here")


# speedup vs baseline: 3.4621x; 3.4621x over previous
import functools

import jax
import jax.numpy as jnp
from jax.experimental import pallas as pl
from jax.experimental.pallas import tpu as pltpu

_BS, _T, _NAG, _NTASK, _L = 32, 32, 20, 50, 4
_M, _N, _m = 8, 8, 50
_HID = 256
_IN = _N * _M * _L + _N * _m + _N + _N * _M
_INP = 768
_ROWS = _BS * _T * _NAG
_RB = 256


def _mlp_body(x_ref, w1_ref, b1_ref, w2_ref, b2_ref, w3_ref, b3_ref, o_ref):
    x = x_ref[...]
    h = jnp.dot(x, w1_ref[...], preferred_element_type=jnp.float32) + b1_ref[...]
    h = jnp.maximum(h, 0.0)
    h = jnp.dot(h, w2_ref[...], preferred_element_type=jnp.float32) + b2_ref[...]
    h = jnp.maximum(h, 0.0)
    o_ref[...] = jnp.dot(h, w3_ref[...], preferred_element_type=jnp.float32) + b3_ref[...]


def _mlp(feats, W1p, b1, W2, b2, W3, b3):
    grid = (_ROWS // _RB,)
    return pl.pallas_call(
        _mlp_body,
        grid=grid,
        in_specs=[
            pl.BlockSpec((_RB, _INP), lambda i: (i, 0)),
            pl.BlockSpec((_INP, _HID), lambda i: (0, 0)),
            pl.BlockSpec((1, _HID), lambda i: (0, 0)),
            pl.BlockSpec((_HID, _HID), lambda i: (0, 0)),
            pl.BlockSpec((1, _HID), lambda i: (0, 0)),
            pl.BlockSpec((_HID, _M + 1), lambda i: (0, 0)),
            pl.BlockSpec((1, _M + 1), lambda i: (0, 0)),
        ],
        out_specs=pl.BlockSpec((_RB, _M + 1), lambda i: (i, 0)),
        out_shape=jax.ShapeDtypeStruct((_ROWS, _M + 1), jnp.float32),
    )(feats, W1p, b1.reshape(1, -1), W2, b2.reshape(1, -1), W3, b3.reshape(1, -1))


def _features_jnp(beta, actions_onehot, power_states, prev_assigns):
    bs, t, n = beta.shape[0], beta.shape[1], beta.shape[2]
    total_beta = beta.sum(axis=-1)
    top_tasks = jax.lax.top_k(total_beta, _M)[1]
    onehot_pa = jax.nn.one_hot(prev_assigns, _m, dtype=jnp.float32)
    feats = []
    for i in range(n):
        top_i = top_tasks[:, :, i, :]
        top_exp = jnp.broadcast_to(top_i[:, :, None, :], (bs, t, n, _M))
        benefits = jnp.take_along_axis(total_beta, top_exp, axis=3)
        best = benefits.max(axis=-1)
        best = best.at[:, :, i].set(-jnp.inf)
        nbr = jax.lax.top_k(best, _N)[1]
        beta_nb = jnp.take_along_axis(beta, nbr[:, :, :, None, None], axis=2)
        beta_g = jnp.take_along_axis(beta_nb, top_i[:, :, None, :, None], axis=3)
        beta_feat = jnp.transpose(beta_g, (0, 1, 3, 2, 4)).reshape(bs, t, _N * _M * _L)
        act_feat = jnp.take_along_axis(actions_onehot, nbr[:, :, :, None], axis=2).reshape(bs, t, _N * _m)
        pw_feat = jnp.take_along_axis(power_states, nbr, axis=2)
        pa_nb = jnp.take_along_axis(onehot_pa, nbr[:, :, :, None], axis=2)
        pa_g = jnp.take_along_axis(pa_nb, top_i[:, :, None, :], axis=3)
        pa_feat = jnp.transpose(pa_g, (0, 1, 3, 2)).reshape(bs, t, _N * _M)
        zeros = jnp.zeros((bs, t, _INP - _IN), jnp.float32)
        feats.append(jnp.concatenate([beta_feat, act_feat, pa_feat, pw_feat, zeros], axis=-1))
    return jnp.stack(feats, axis=2).reshape(_ROWS, _INP)


def kernel(beta, actions_onehot, power_states, prev_assigns, W1, b1, W2, b2, W3, b3):
    feats = _features_jnp(beta, actions_onehot, power_states, prev_assigns)
    W1p = jnp.concatenate(
        [W1[: 656], W1[664:728], W1[656:664], jnp.zeros((_INP - _IN, _HID), jnp.float32)], axis=0
    )
    q = _mlp(feats, W1p, b1, W2, b2, W3, b3)
    return q.reshape(_BS, _T, _NAG, _M + 1)


# speedup vs baseline: 26.3328x; 7.6061x over previous
import functools

import jax
import jax.numpy as jnp
from jax import lax
from jax.experimental import pallas as pl
from jax.experimental.pallas import tpu as pltpu
from jax.experimental.pallas import tpu_sc as plsc

_BS, _T, _NAG, _NTASK, _L = 32, 32, 20, 50, 4
_M, _N, _m = 8, 8, 50
_HID = 256
_IN = _N * _M * _L + _N * _m + _N + _N * _M
_INP = 768
_PAIRS = _BS * _T
_ROWS = _PAIRS * _NAG
_RB = 256
_TBW = 64
_NEG = float("-inf")
_BIG = 1 << 20


def _vgather(x, idx):
    return x.at[idx].get(mode="promise_in_bounds")


def _splat_lane(x, k):
    return _vgather(x, jnp.full((16,), k, jnp.int32))


def _vmax_all(x):
    lane = lax.iota(jnp.int32, 16)
    for d in (8, 4, 2, 1):
        x = jnp.maximum(x, _vgather(x, jnp.bitwise_xor(lane, d)))
    return x


def _vmin_all(x):
    lane = lax.iota(jnp.int32, 16)
    for d in (8, 4, 2, 1):
        x = jnp.minimum(x, _vgather(x, jnp.bitwise_xor(lane, d)))
    return x


def _sc_body(beta_h, tb_h, act_h, pw_h, prev_h, out_h,
             beta_v, tb_v, act_v, pw_v, prev_v, out_v):
    ncores = 2
    wid = lax.axis_index("s") * ncores + lax.axis_index("c")
    lane = lax.iota(jnp.int32, 16)
    pairs_per = _PAIRS // 32

    def per_pair(p, carry):
        pair = wid * pairs_per + p
        pltpu.sync_copy(beta_h.at[pair], beta_v)
        pltpu.sync_copy(tb_h.at[pair], tb_v)
        pltpu.sync_copy(act_h.at[pair], act_v)
        pltpu.sync_copy(pw_h.at[pair], pw_v)
        pltpu.sync_copy(prev_h.at[pair], prev_v)

        def per_agent(i, carry2):
            base = i * _INP
            v0 = tb_v[pl.ds(i * _TBW, 16)]
            v1 = tb_v[pl.ds(i * _TBW + 16, 16)]
            v2 = tb_v[pl.ds(i * _TBW + 32, 16)]
            v3 = tb_v[pl.ds(i * _TBW + 48, 16)]
            top0 = jnp.zeros((16,), jnp.int32)

            def task_iter(it, st):
                a0, a1, a2, a3, top = st
                s = _vmax_all(jnp.maximum(jnp.maximum(a0, a1), jnp.maximum(a2, a3)))
                c0 = jnp.where(a0 == s, lane, _BIG)
                c1 = jnp.where(a1 == s, lane + 16, _BIG)
                c2 = jnp.where(a2 == s, lane + 32, _BIG)
                c3 = jnp.where(a3 == s, lane + 48, _BIG)
                cmin = _vmin_all(jnp.minimum(jnp.minimum(c0, c1), jnp.minimum(c2, c3)))
                top = jnp.where(lane == it, cmin, top)
                a0 = jnp.where(lane == cmin, _NEG, a0)
                a1 = jnp.where(lane + 16 == cmin, _NEG, a1)
                a2 = jnp.where(lane + 32 == cmin, _NEG, a2)
                a3 = jnp.where(lane + 48 == cmin, _NEG, a3)
                return a0, a1, a2, a3, top

            _, _, _, _, top = lax.fori_loop(0, _M, task_iter, (v0, v1, v2, v3, top0))
            tcol = [_splat_lane(top, mm) for mm in range(_M)]

            j0 = lane
            j1 = lane + 16
            f0 = j0 * _TBW
            f1 = j1 * _TBW
            b0 = plsc.load_gather(tb_v, [f0 + tcol[0]])
            b1 = plsc.load_gather(tb_v, [jnp.minimum(f1 + tcol[0], _PAD_TB - 1)])
            for mm in range(1, _M):
                b0 = jnp.maximum(b0, plsc.load_gather(tb_v, [f0 + tcol[mm]]))
                b1 = jnp.maximum(
                    b1, plsc.load_gather(tb_v, [jnp.minimum(f1 + tcol[mm], _PAD_TB - 1)]))
            b0 = jnp.where(j0 == i, _NEG, b0)
            b1 = jnp.where(j1 == i, _NEG, b1)
            b1 = jnp.where(j1 >= _NAG, _NEG, b1)

            nbr0 = jnp.zeros((16,), jnp.int32)

            def nbr_iter(it, st):
                a0, a1, nbr = st
                s = _vmax_all(jnp.maximum(a0, a1))
                c0 = jnp.where(a0 == s, lane, _BIG)
                c1 = jnp.where(a1 == s, lane + 16, _BIG)
                cmin = _vmin_all(jnp.minimum(c0, c1))
                nbr = jnp.where(lane == it, cmin, nbr)
                a0 = jnp.where(lane == cmin, _NEG, a0)
                a1 = jnp.where(lane + 16 == cmin, _NEG, a1)
                return a0, a1, nbr

            _, _, nbr = lax.fori_loop(0, _N, nbr_iter, (b0, b1, nbr0))

            for v in range(16):
                p = lane + 16 * v
                nbi = jnp.bitwise_and(lax.shift_right_logical(p, 2), 7)
                ll = jnp.bitwise_and(p, 3)
                row = _vgather(nbr, nbi)
                flat = row * (_NTASK * _L) + tcol[v // 2] * _L + ll
                out_v[pl.ds(base + 16 * v, 16)] = plsc.load_gather(beta_v, [flat])
            for v in range(25):
                p = lane + 16 * v
                nbi = p // _m
                k = p - nbi * _m
                row = _vgather(nbr, nbi)
                out_v[pl.ds(base + 256 + 16 * v, 16)] = plsc.load_gather(
                    act_v, [row * _m + k])
            for v in range(4):
                p = lane + 16 * v
                mi = lax.shift_right_logical(p, 3)
                nbi = jnp.bitwise_and(p, 7)
                a = _vgather(nbr, nbi)
                pv = plsc.load_gather(prev_v, [a])
                tt = _vgather(top, mi)
                out_v[pl.ds(base + 656 + 16 * v, 16)] = jnp.where(pv == tt, 1.0, 0.0)
            a = _vgather(nbr, jnp.where(lane < 8, lane, 0))
            pwv = plsc.load_gather(pw_v, [a])
            out_v[pl.ds(base + 720, 16)] = jnp.where(lane < 8, pwv, 0.0)
            out_v[pl.ds(base + 736, 16)] = jnp.zeros((16,), jnp.float32)
            out_v[pl.ds(base + 752, 16)] = jnp.zeros((16,), jnp.float32)
            return carry2

        lax.fori_loop(0, _NAG, per_agent, 0)
        pltpu.sync_copy(out_v, out_h.at[pair])
        return carry

    lax.fori_loop(0, pairs_per, per_pair, 0)


_PAD_TB = _NAG * _TBW


def _sc_features(beta_f, tbp, act, pwp, prevp):
    mesh = plsc.VectorSubcoreMesh(
        core_axis_name="c", subcore_axis_name="s", num_cores=2, num_subcores=16)
    return pl.kernel(
        _sc_body,
        out_type=jax.ShapeDtypeStruct((_PAIRS, _NAG * _INP), jnp.float32),
        mesh=mesh,
        compiler_params=pltpu.CompilerParams(needs_layout_passes=False),
        scratch_types=[
            pltpu.VMEM((_NAG * _NTASK * _L,), jnp.float32),
            pltpu.VMEM((_PAD_TB,), jnp.float32),
            pltpu.VMEM((_NAG * _m,), jnp.float32),
            pltpu.VMEM((32,), jnp.float32),
            pltpu.VMEM((32,), jnp.int32),
            pltpu.VMEM((_NAG * _INP,), jnp.float32),
        ],
    )(beta_f, tbp, act, pwp, prevp)


def _mlp_body(x_ref, w1_ref, b1_ref, w2_ref, b2_ref, w3_ref, b3_ref, o_ref):
    x = x_ref[...]
    h = jnp.dot(x, w1_ref[...], preferred_element_type=jnp.float32) + b1_ref[...]
    h = jnp.maximum(h, 0.0)
    h = jnp.dot(h, w2_ref[...], preferred_element_type=jnp.float32) + b2_ref[...]
    h = jnp.maximum(h, 0.0)
    o_ref[...] = jnp.dot(h, w3_ref[...], preferred_element_type=jnp.float32) + b3_ref[...]


def _mlp(feats, W1p, b1, W2, b2, W3, b3):
    grid = (_ROWS // _RB,)
    return pl.pallas_call(
        _mlp_body,
        grid=grid,
        in_specs=[
            pl.BlockSpec((_RB, _INP), lambda i: (i, 0)),
            pl.BlockSpec((_INP, _HID), lambda i: (0, 0)),
            pl.BlockSpec((1, _HID), lambda i: (0, 0)),
            pl.BlockSpec((_HID, _HID), lambda i: (0, 0)),
            pl.BlockSpec((1, _HID), lambda i: (0, 0)),
            pl.BlockSpec((_HID, _M + 1), lambda i: (0, 0)),
            pl.BlockSpec((1, _M + 1), lambda i: (0, 0)),
        ],
        out_specs=pl.BlockSpec((_RB, _M + 1), lambda i: (i, 0)),
        out_shape=jax.ShapeDtypeStruct((_ROWS, _M + 1), jnp.float32),
    )(feats, W1p, b1.reshape(1, -1), W2, b2.reshape(1, -1), W3, b3.reshape(1, -1))


def kernel(beta, actions_onehot, power_states, prev_assigns, W1, b1, W2, b2, W3, b3):
    total_beta = beta.sum(axis=-1)
    tbp = jnp.pad(total_beta.reshape(_PAIRS, _NAG, _NTASK),
                  ((0, 0), (0, 0), (0, _TBW - _NTASK)),
                  constant_values=_NEG).reshape(_PAIRS, _PAD_TB)
    beta_f = beta.reshape(_PAIRS, _NAG * _NTASK * _L)
    act = actions_onehot.reshape(_PAIRS, _NAG * _m)
    pwp = jnp.pad(power_states.reshape(_PAIRS, _NAG), ((0, 0), (0, 12)))
    prevp = jnp.pad(prev_assigns.astype(jnp.int32).reshape(_PAIRS, _NAG),
                    ((0, 0), (0, 12)))
    feats = _sc_features(beta_f, tbp, act, pwp, prevp).reshape(_ROWS, _INP)
    W1p = jnp.concatenate(
        [W1[:656], W1[664:728], W1[656:664], jnp.zeros((_INP - _IN, _HID), jnp.float32)],
        axis=0)
    q = _mlp(feats, W1p, b1, W2, b2, W3, b3)
    return q.reshape(_BS, _T, _NAG, _M + 1)


# speedup vs baseline: 31.6195x; 1.2008x over previous
import functools

import jax
import jax.numpy as jnp
from jax import lax
from jax.experimental import pallas as pl
from jax.experimental.pallas import tpu as pltpu
from jax.experimental.pallas import tpu_sc as plsc

_BS, _T, _NAG, _NTASK, _L = 32, 32, 20, 50, 4
_M, _N, _m = 8, 8, 50
_HID = 256
_IN = _N * _M * _L + _N * _m + _N + _N * _M
_INP = 768
_PAIRS = _BS * _T
_ROWS = _PAIRS * _NAG
_RB = 256
_NEG = float("-inf")
_BIG = 1 << 20
_CW = 2064
_ACT0, _PW0, _PREV0 = 1000, 2000, 2032
_NW = 32
_PPW = _PAIRS // _NW


def _vgather(x, idx):
    return x.at[idx].get(mode="promise_in_bounds")


def _splat_lane(x, k):
    return _vgather(x, jnp.full((16,), k, jnp.int32))


def _vmax_all(x):
    lane = lax.iota(jnp.int32, 16)
    for d in (8, 4, 2, 1):
        x = jnp.maximum(x, _vgather(x, jnp.bitwise_xor(lane, d)))
    return x


def _vmin_all(x):
    lane = lax.iota(jnp.int32, 16)
    for d in (8, 4, 2, 1):
        x = jnp.minimum(x, _vgather(x, jnp.bitwise_xor(lane, d)))
    return x


def _sc_body(beta_h, comb_h, out_h,
             beta_v0, comb_v0, out_v0, beta_v1, comb_v1, out_v1,
             isem0, isem1, osem0, osem1):
    lane = lax.iota(jnp.int32, 16)
    wid = lax.axis_index("s") * 2 + lax.axis_index("c")
    base = wid * _PPW
    beta_v = (beta_v0, beta_v1)
    comb_v = (comb_v0, comb_v1)
    out_v = (out_v0, out_v1)
    isem = (isem0, isem1)
    osem = (osem0, osem1)

    def fire_in(pair, b):
        pltpu.async_copy(beta_h.at[pair], beta_v[b], isem[b])
        pltpu.async_copy(comb_h.at[pair], comb_v[b], isem[b])

    def wait_in(pair, b):
        pltpu.make_async_copy(beta_h.at[pair], beta_v[b], isem[b]).wait()
        pltpu.make_async_copy(comb_h.at[pair], comb_v[b], isem[b]).wait()

    def compute(pair, b):
        cv = comb_v[b]
        bv = beta_v[b]
        ov = out_v[b]

        def per_agent(i, carry2):
            rowbase = i * _INP
            v0 = cv[pl.ds(i * _NTASK, 16)]
            v1 = cv[pl.ds(i * _NTASK + 16, 16)]
            v2 = cv[pl.ds(i * _NTASK + 32, 16)]
            v3 = cv[pl.ds(i * _NTASK + 48, 16)]
            v3 = jnp.where(lane + 48 < _NTASK, v3, _NEG)
            top0 = jnp.zeros((16,), jnp.int32)

            def task_iter(it, st):
                a0, a1, a2, a3, top = st
                s = _vmax_all(jnp.maximum(jnp.maximum(a0, a1), jnp.maximum(a2, a3)))
                c0 = jnp.where(a0 == s, lane, _BIG)
                c1 = jnp.where(a1 == s, lane + 16, _BIG)
                c2 = jnp.where(a2 == s, lane + 32, _BIG)
                c3 = jnp.where(a3 == s, lane + 48, _BIG)
                cmin = _vmin_all(jnp.minimum(jnp.minimum(c0, c1), jnp.minimum(c2, c3)))
                top = jnp.where(lane == it, cmin, top)
                a0 = jnp.where(lane == cmin, _NEG, a0)
                a1 = jnp.where(lane + 16 == cmin, _NEG, a1)
                a2 = jnp.where(lane + 32 == cmin, _NEG, a2)
                a3 = jnp.where(lane + 48 == cmin, _NEG, a3)
                return a0, a1, a2, a3, top

            _, _, _, _, top = lax.fori_loop(0, _M, task_iter, (v0, v1, v2, v3, top0))
            tcol = [_splat_lane(top, mm) for mm in range(_M)]

            j0 = lane
            j1 = lane + 16
            f0 = j0 * _NTASK
            f1 = j1 * _NTASK
            b0 = plsc.load_gather(cv, [f0 + tcol[0]])
            b1 = plsc.load_gather(cv, [f1 + tcol[0]])
            for mm in range(1, _M):
                b0 = jnp.maximum(b0, plsc.load_gather(cv, [f0 + tcol[mm]]))
                b1 = jnp.maximum(b1, plsc.load_gather(cv, [f1 + tcol[mm]]))
            b0 = jnp.where(j0 == i, _NEG, b0)
            b1 = jnp.where(j1 == i, _NEG, b1)
            b1 = jnp.where(j1 >= _NAG, _NEG, b1)

            nbr0 = jnp.zeros((16,), jnp.int32)

            def nbr_iter(it, st):
                a0, a1, nbr = st
                s = _vmax_all(jnp.maximum(a0, a1))
                c0 = jnp.where(a0 == s, lane, _BIG)
                c1 = jnp.where(a1 == s, lane + 16, _BIG)
                cmin = _vmin_all(jnp.minimum(c0, c1))
                nbr = jnp.where(lane == it, cmin, nbr)
                a0 = jnp.where(lane == cmin, _NEG, a0)
                a1 = jnp.where(lane + 16 == cmin, _NEG, a1)
                return a0, a1, nbr

            _, _, nbr = lax.fori_loop(0, _N, nbr_iter, (b0, b1, nbr0))

            for v in range(16):
                p = lane + 16 * v
                nbi = jnp.bitwise_and(lax.shift_right_logical(p, 2), 7)
                ll = jnp.bitwise_and(p, 3)
                row = _vgather(nbr, nbi)
                flat = row * (_NTASK * _L) + tcol[v // 2] * _L + ll
                ov[pl.ds(rowbase + 16 * v, 16)] = plsc.load_gather(bv, [flat])
            for v in range(25):
                p = lane + 16 * v
                nbi = lax.div(p, _m)
                k = p - nbi * _m
                row = _vgather(nbr, nbi)
                ov[pl.ds(rowbase + 256 + 16 * v, 16)] = plsc.load_gather(
                    cv, [_ACT0 + row * _m + k])
            for v in range(4):
                p = lane + 16 * v
                mi = lax.shift_right_logical(p, 3)
                nbi = jnp.bitwise_and(p, 7)
                a = _vgather(nbr, nbi)
                pv = plsc.bitcast(plsc.load_gather(cv, [_PREV0 + a]), jnp.int32)
                tt = _vgather(top, mi)
                ov[pl.ds(rowbase + 656 + 16 * v, 16)] = jnp.where(pv == tt, 1.0, 0.0)
            a = _vgather(nbr, jnp.where(lane < 8, lane, 0))
            pwv = plsc.load_gather(cv, [_PW0 + a])
            ov[pl.ds(rowbase + 720, 16)] = jnp.where(lane < 8, pwv, 0.0)
            ov[pl.ds(rowbase + 736, 16)] = jnp.zeros((16,), jnp.float32)
            ov[pl.ds(rowbase + 752, 16)] = jnp.zeros((16,), jnp.float32)
            return carry2

        lax.fori_loop(0, _NAG, per_agent, 0)

    fire_in(base, 0)

    def outer(g, carry):
        for bb in range(2):
            p = g * 2 + bb
            pair = base + p

            @pl.when(p + 1 < _PPW)
            def _():
                fire_in(pair + 1, 1 - bb)

            wait_in(pair, bb)

            @pl.when(p >= 2)
            def _():
                pltpu.make_async_copy(out_v[bb], out_h.at[pair], osem[bb]).wait()

            compute(pair, bb)
            pltpu.async_copy(out_v[bb], out_h.at[pair], osem[bb])
        return carry

    lax.fori_loop(0, _PPW // 2, outer, 0)
    pltpu.make_async_copy(out_v[0], out_h.at[base], osem[0]).wait()
    pltpu.make_async_copy(out_v[1], out_h.at[base], osem[1]).wait()


def _sc_features(beta_f, comb):
    mesh = plsc.VectorSubcoreMesh(
        core_axis_name="c", subcore_axis_name="s", num_cores=2, num_subcores=16)
    return pl.kernel(
        _sc_body,
        out_type=jax.ShapeDtypeStruct((_PAIRS, _NAG * _INP), jnp.float32),
        mesh=mesh,
        compiler_params=pltpu.CompilerParams(needs_layout_passes=False),
        scratch_types=[
            pltpu.VMEM((_NAG * _NTASK * _L,), jnp.float32),
            pltpu.VMEM((_CW,), jnp.float32),
            pltpu.VMEM((_NAG * _INP,), jnp.float32),
            pltpu.VMEM((_NAG * _NTASK * _L,), jnp.float32),
            pltpu.VMEM((_CW,), jnp.float32),
            pltpu.VMEM((_NAG * _INP,), jnp.float32),
            pltpu.SemaphoreType.DMA,
            pltpu.SemaphoreType.DMA,
            pltpu.SemaphoreType.DMA,
            pltpu.SemaphoreType.DMA,
        ],
    )(beta_f, comb)


def _mlp_body(x_ref, w1_ref, b1_ref, w2_ref, b2_ref, w3_ref, b3_ref, o_ref):
    x = x_ref[...]
    h = jnp.dot(x, w1_ref[...], preferred_element_type=jnp.float32) + b1_ref[...]
    h = jnp.maximum(h, 0.0)
    h = jnp.dot(h, w2_ref[...], preferred_element_type=jnp.float32) + b2_ref[...]
    h = jnp.maximum(h, 0.0)
    o_ref[...] = jnp.dot(h, w3_ref[...], preferred_element_type=jnp.float32) + b3_ref[...]


def _mlp(feats, W1p, b1, W2, b2, W3, b3):
    grid = (_ROWS // _RB,)
    return pl.pallas_call(
        _mlp_body,
        grid=grid,
        in_specs=[
            pl.BlockSpec((_RB, _INP), lambda i: (i, 0)),
            pl.BlockSpec((_INP, _HID), lambda i: (0, 0)),
            pl.BlockSpec((1, _HID), lambda i: (0, 0)),
            pl.BlockSpec((_HID, _HID), lambda i: (0, 0)),
            pl.BlockSpec((1, _HID), lambda i: (0, 0)),
            pl.BlockSpec((_HID, _M + 1), lambda i: (0, 0)),
            pl.BlockSpec((1, _M + 1), lambda i: (0, 0)),
        ],
        out_specs=pl.BlockSpec((_RB, _M + 1), lambda i: (i, 0)),
        out_shape=jax.ShapeDtypeStruct((_ROWS, _M + 1), jnp.float32),
    )(feats, W1p, b1.reshape(1, -1), W2, b2.reshape(1, -1), W3, b3.reshape(1, -1))


def kernel(beta, actions_onehot, power_states, prev_assigns, W1, b1, W2, b2, W3, b3):
    total_beta = beta.sum(axis=-1)
    beta_f = beta.reshape(_PAIRS, _NAG * _NTASK * _L)
    comb = jnp.concatenate([
        total_beta.reshape(_PAIRS, _NAG * _NTASK),
        actions_onehot.reshape(_PAIRS, _NAG * _m),
        jnp.pad(power_states.reshape(_PAIRS, _NAG), ((0, 0), (0, 12))),
        jnp.pad(jax.lax.bitcast_convert_type(
            prev_assigns.astype(jnp.int32), jnp.float32).reshape(_PAIRS, _NAG),
            ((0, 0), (0, 12))),
    ], axis=1)
    feats = _sc_features(beta_f, comb).reshape(_ROWS, _INP)
    W1p = jnp.concatenate(
        [W1[:656], W1[664:728], W1[656:664], jnp.zeros((_INP - _IN, _HID), jnp.float32)],
        axis=0)
    q = _mlp(feats, W1p, b1, W2, b2, W3, b3)
    return q.reshape(_BS, _T, _NAG, _M + 1)


# speedup vs baseline: 31.8993x; 1.0088x over previous
import functools

import jax
import jax.numpy as jnp
from jax import lax
from jax.experimental import pallas as pl
from jax.experimental.pallas import tpu as pltpu
from jax.experimental.pallas import tpu_sc as plsc

_BS, _T, _NAG, _NTASK, _L = 32, 32, 20, 50, 4
_M, _N, _m = 8, 8, 50
_HID = 256
_IN = _N * _M * _L + _N * _m + _N + _N * _M
_INP = 768
_PAIRS = _BS * _T
_ROWS = _PAIRS * _NAG
_RB = 256
_NEG = float("-inf")
_BIG = 1 << 20
_CW = 2064
_ACT0, _PW0, _PREV0 = 1000, 2000, 2032
_NW = 32
_PPW = _PAIRS // _NW


def _vgather(x, idx):
    return x.at[idx].get(mode="promise_in_bounds")


def _splat_lane(x, k):
    return _vgather(x, jnp.full((16,), k, jnp.int32))


def _vmax_all(x):
    lane = lax.iota(jnp.int32, 16)
    for d in (8, 4, 2, 1):
        x = jnp.maximum(x, _vgather(x, jnp.bitwise_xor(lane, d)))
    return x


def _vmin_all(x):
    lane = lax.iota(jnp.int32, 16)
    for d in (8, 4, 2, 1):
        x = jnp.minimum(x, _vgather(x, jnp.bitwise_xor(lane, d)))
    return x


def _sc_body(beta_h, tb_h, act_h, pw_h, prev_h, out_h,
             beta_v0, tb_v0, act_v0, pw_v0, prev_v0, out_v0,
             beta_v1, tb_v1, act_v1, pw_v1, prev_v1, out_v1,
             isem0, isem1, osem0, osem1):
    lane = lax.iota(jnp.int32, 16)
    wid = lax.axis_index("s") * 2 + lax.axis_index("c")
    base = wid * _PPW
    beta_v = (beta_v0, beta_v1)
    tb_v = (tb_v0, tb_v1)
    act_v = (act_v0, act_v1)
    pw_v = (pw_v0, pw_v1)
    prev_v = (prev_v0, prev_v1)
    out_v = (out_v0, out_v1)
    isem = (isem0, isem1)
    osem = (osem0, osem1)

    def _in_copies(pair, b):
        return (
            pltpu.make_async_copy(beta_h.at[pair], beta_v[b], isem[b]),
            pltpu.make_async_copy(tb_h.at[pair], tb_v[b], isem[b]),
            pltpu.make_async_copy(act_h.at[pair], act_v[b], isem[b]),
            pltpu.make_async_copy(pw_h.at[pair], pw_v[b], isem[b]),
            pltpu.make_async_copy(prev_h.at[pair], prev_v[b], isem[b]),
        )

    def fire_in(pair, b):
        for c in _in_copies(pair, b):
            c.start()

    def wait_in(pair, b):
        for c in _in_copies(pair, b):
            c.wait()

    def compute(pair, b):
        tv = tb_v[b]
        av = act_v[b]
        bv = beta_v[b]
        ov = out_v[b]

        def per_agent(i, carry2):
            rowbase = i * _INP
            v0 = tv[pl.ds(i * _NTASK, 16)]
            v1 = tv[pl.ds(i * _NTASK + 16, 16)]
            v2 = tv[pl.ds(i * _NTASK + 32, 16)]
            v3 = tv[pl.ds(i * _NTASK + 34, 16)]
            top0 = jnp.zeros((16,), jnp.int32)

            def task_iter(it, st):
                a0, a1, a2, a3, top = st
                s = _vmax_all(jnp.maximum(jnp.maximum(a0, a1), jnp.maximum(a2, a3)))
                c0 = jnp.where(a0 == s, lane, _BIG)
                c1 = jnp.where(a1 == s, lane + 16, _BIG)
                c2 = jnp.where(a2 == s, lane + 32, _BIG)
                c3 = jnp.where(a3 == s, lane + 34, _BIG)
                cmin = _vmin_all(jnp.minimum(jnp.minimum(c0, c1), jnp.minimum(c2, c3)))
                top = jnp.where(lane == it, cmin, top)
                a0 = jnp.where(lane == cmin, _NEG, a0)
                a1 = jnp.where(lane + 16 == cmin, _NEG, a1)
                a2 = jnp.where(lane + 32 == cmin, _NEG, a2)
                a3 = jnp.where(lane + 34 == cmin, _NEG, a3)
                return a0, a1, a2, a3, top

            _, _, _, _, top = lax.fori_loop(0, _M, task_iter, (v0, v1, v2, v3, top0))
            tcol = [_splat_lane(top, mm) for mm in range(_M)]

            j0 = lane
            j1 = lane + 16
            f0 = j0 * _NTASK
            f1 = jnp.minimum(j1, _NAG - 1) * _NTASK
            b0 = plsc.load_gather(tv, [f0 + tcol[0]])
            b1 = plsc.load_gather(tv, [f1 + tcol[0]])
            for mm in range(1, _M):
                b0 = jnp.maximum(b0, plsc.load_gather(tv, [f0 + tcol[mm]]))
                b1 = jnp.maximum(b1, plsc.load_gather(tv, [f1 + tcol[mm]]))
            b0 = jnp.where(j0 == i, _NEG, b0)
            b1 = jnp.where(j1 == i, _NEG, b1)
            b1 = jnp.where(j1 >= _NAG, _NEG, b1)

            nbr0 = jnp.zeros((16,), jnp.int32)

            def nbr_iter(it, st):
                a0, a1, nbr = st
                s = _vmax_all(jnp.maximum(a0, a1))
                c0 = jnp.where(a0 == s, lane, _BIG)
                c1 = jnp.where(a1 == s, lane + 16, _BIG)
                cmin = _vmin_all(jnp.minimum(c0, c1))
                nbr = jnp.where(lane == it, cmin, nbr)
                a0 = jnp.where(lane == cmin, _NEG, a0)
                a1 = jnp.where(lane + 16 == cmin, _NEG, a1)
                return a0, a1, nbr

            _, _, nbr = lax.fori_loop(0, _N, nbr_iter, (b0, b1, nbr0))

            for v in range(16):
                p = lane + 16 * v
                nbi = jnp.bitwise_and(lax.shift_right_logical(p, 2), 7)
                ll = jnp.bitwise_and(p, 3)
                row = _vgather(nbr, nbi)
                flat = row * (_NTASK * _L) + tcol[v // 2] * _L + ll
                ov[pl.ds(rowbase + 16 * v, 16)] = plsc.load_gather(bv, [flat])
            for v in range(25):
                p = lane + 16 * v
                nbi = lax.div(p, _m)
                k = p - nbi * _m
                row = _vgather(nbr, nbi)
                ov[pl.ds(rowbase + 256 + 16 * v, 16)] = plsc.load_gather(
                    av, [row * _m + k])
            for v in range(4):
                p = lane + 16 * v
                mi = lax.shift_right_logical(p, 3)
                nbi = jnp.bitwise_and(p, 7)
                a = _vgather(nbr, nbi)
                pv = plsc.bitcast(plsc.load_gather(prev_v[b], [a]), jnp.int32)
                tt = _vgather(top, mi)
                ov[pl.ds(rowbase + 656 + 16 * v, 16)] = jnp.where(pv == tt, 1.0, 0.0)
            a = _vgather(nbr, jnp.where(lane < 8, lane, 0))
            pwv = plsc.load_gather(pw_v[b], [a])
            ov[pl.ds(rowbase + 720, 16)] = jnp.where(lane < 8, pwv, 0.0)
            ov[pl.ds(rowbase + 736, 16)] = jnp.zeros((16,), jnp.float32)
            ov[pl.ds(rowbase + 752, 16)] = jnp.zeros((16,), jnp.float32)
            return carry2

        lax.fori_loop(0, _NAG, per_agent, 0)

    fire_in(base, 0)

    def outer(g, carry):
        for bb in range(2):
            p = g * 2 + bb
            pair = base + p

            @pl.when(p + 1 < _PPW)
            def _():
                fire_in(pair + 1, 1 - bb)

            wait_in(pair, bb)

            @pl.when(p >= 2)
            def _():
                pltpu.make_async_copy(out_v[bb], out_h.at[pair], osem[bb]).wait()

            compute(pair, bb)
            pltpu.async_copy(out_v[bb], out_h.at[pair], osem[bb])
        return carry

    lax.fori_loop(0, _PPW // 2, outer, 0)
    pltpu.make_async_copy(out_v[0], out_h.at[base], osem[0]).wait()
    pltpu.make_async_copy(out_v[1], out_h.at[base], osem[1]).wait()


def _sc_features(beta_f, tb, act, pw, prev):
    mesh = plsc.VectorSubcoreMesh(
        core_axis_name="c", subcore_axis_name="s", num_cores=2, num_subcores=16)
    return pl.kernel(
        _sc_body,
        out_type=jax.ShapeDtypeStruct((_PAIRS, _NAG * _INP), jnp.float32),
        mesh=mesh,
        compiler_params=pltpu.CompilerParams(needs_layout_passes=False),
        scratch_types=[
            pltpu.VMEM((_NAG * _NTASK * _L,), jnp.float32),
            pltpu.VMEM((_NAG * _NTASK,), jnp.float32),
            pltpu.VMEM((_NAG * _m,), jnp.float32),
            pltpu.VMEM((32,), jnp.float32),
            pltpu.VMEM((32,), jnp.float32),
            pltpu.VMEM((_NAG * _INP,), jnp.float32),
            pltpu.VMEM((_NAG * _NTASK * _L,), jnp.float32),
            pltpu.VMEM((_NAG * _NTASK,), jnp.float32),
            pltpu.VMEM((_NAG * _m,), jnp.float32),
            pltpu.VMEM((32,), jnp.float32),
            pltpu.VMEM((32,), jnp.float32),
            pltpu.VMEM((_NAG * _INP,), jnp.float32),
            pltpu.SemaphoreType.DMA,
            pltpu.SemaphoreType.DMA,
            pltpu.SemaphoreType.DMA,
            pltpu.SemaphoreType.DMA,
        ],
    )(beta_f, tb, act, pw, prev)


def _mlp_body(x_ref, w1_ref, b1_ref, w2_ref, b2_ref, w3_ref, b3_ref, o_ref):
    x = x_ref[...]
    h = jnp.dot(x, w1_ref[...], preferred_element_type=jnp.float32) + b1_ref[...]
    h = jnp.maximum(h, 0.0)
    h = jnp.dot(h, w2_ref[...], preferred_element_type=jnp.float32) + b2_ref[...]
    h = jnp.maximum(h, 0.0)
    o_ref[...] = jnp.dot(h, w3_ref[...], preferred_element_type=jnp.float32) + b3_ref[...]


def _mlp(feats, W1p, b1, W2, b2, W3, b3):
    grid = (_ROWS // _RB,)
    return pl.pallas_call(
        _mlp_body,
        grid=grid,
        in_specs=[
            pl.BlockSpec((_RB, _INP), lambda i: (i, 0)),
            pl.BlockSpec((_INP, _HID), lambda i: (0, 0)),
            pl.BlockSpec((1, _HID), lambda i: (0, 0)),
            pl.BlockSpec((_HID, _HID), lambda i: (0, 0)),
            pl.BlockSpec((1, _HID), lambda i: (0, 0)),
            pl.BlockSpec((_HID, _M + 1), lambda i: (0, 0)),
            pl.BlockSpec((1, _M + 1), lambda i: (0, 0)),
        ],
        out_specs=pl.BlockSpec((_RB, _M + 1), lambda i: (i, 0)),
        out_shape=jax.ShapeDtypeStruct((_ROWS, _M + 1), jnp.float32),
    )(feats, W1p, b1.reshape(1, -1), W2, b2.reshape(1, -1), W3, b3.reshape(1, -1))


def kernel(beta, actions_onehot, power_states, prev_assigns, W1, b1, W2, b2, W3, b3):
    total_beta = beta.sum(axis=-1)
    beta_f = beta.reshape(_PAIRS, _NAG * _NTASK * _L)
    tb = total_beta.reshape(_PAIRS, _NAG * _NTASK)
    act = actions_onehot.reshape(_PAIRS, _NAG * _m)
    pw = jnp.pad(power_states.reshape(_PAIRS, _NAG), ((0, 0), (0, 12)))
    prev = jnp.pad(jax.lax.bitcast_convert_type(
        prev_assigns.astype(jnp.int32), jnp.float32).reshape(_PAIRS, _NAG),
        ((0, 0), (0, 12)))
    feats = _sc_features(beta_f, tb, act, pw, prev).reshape(_ROWS, _INP)
    W1p = jnp.concatenate(
        [W1[:656], W1[664:728], W1[656:664], jnp.zeros((_INP - _IN, _HID), jnp.float32)],
        axis=0)
    q = _mlp(feats, W1p, b1, W2, b2, W3, b3)
    return q.reshape(_BS, _T, _NAG, _M + 1)
